# Initial kernel scaffold; baseline (speedup 1.0000x reference)
import jax
import jax.numpy as jnp
from jax.experimental import pallas as pl


def kernel(xyz, W0, b0, gamma0, beta0, mean0, var0, W1, b1, gamma1, beta1, mean1, var1):
    raise NotImplementedError("write your pallas kernel here")


# speedup vs baseline: 23.3135x; 23.3135x over previous
import functools

import jax
import jax.numpy as jnp
from jax import lax
from jax.experimental import pallas as pl
from jax.experimental.pallas import tpu as pltpu
from jax.experimental.pallas import tpu_sc as plsc

B = 4
N = 8192
NPOINT = 1024
NSAMPLE = 32
RADIUS = 0.1
EPS = 1e-5

_NTILES = 32
_TILES_PER_BATCH = _NTILES // B
_CENT_PER_TILE = NPOINT // _TILES_PER_BATCH
_GROUPS_PER_TILE = _CENT_PER_TILE // 16
_LIST_CAP = 32


def _fps_body(x_ref, out_ref):
    P = x_ref[...]
    iota = lax.broadcasted_iota(jnp.int32, (B, N), 1)

    def step(k, carry):
        dist, far = carry
        onehot = iota == far
        c = jnp.sum(jnp.where(onehot[None], P, 0.0), axis=2)
        out_ref[pl.ds(k, 1)] = c[None]
        d3 = (P - c[:, :, None]) ** 2
        d = d3[0] + d3[1] + d3[2]
        dist = jnp.minimum(dist, d)
        mx = jnp.max(dist, axis=1, keepdims=True)
        far_new = jnp.min(jnp.where(dist == mx, iota, N), axis=1, keepdims=True)
        return dist, far_new

    dist0 = jnp.full((B, N), jnp.inf, dtype=jnp.float32)
    far0 = jnp.zeros((B, 1), dtype=jnp.int32)
    lax.fori_loop(0, NPOINT, step, (dist0, far0))


def _fps(xyz_cbn):
    return pl.pallas_call(
        _fps_body,
        out_shape=jax.ShapeDtypeStruct((NPOINT, 3, B), jnp.float32),
    )(xyz_cbn)


def _ball_group_body(xyz_hbm, cent_hbm, out_hbm, xv, cv, ov, lv):
    r2 = jnp.float32(RADIUS * RADIUS)
    wid = lax.axis_index("s") * 2 + lax.axis_index("c")
    b = wid // _TILES_PER_BATCH
    t = wid % _TILES_PER_BATCH

    pltpu.sync_copy(xyz_hbm.at[b], xv)
    pltpu.sync_copy(cent_hbm.at[b, :, pl.ds(t * _CENT_PER_TILE, _CENT_PER_TILE)], cv)

    iota16 = lax.iota(jnp.int32, 16)
    zeros16 = jnp.zeros((16,), jnp.int32)
    ones16 = jnp.full((16,), 1, jnp.int32)
    twos16 = jnp.full((16,), 2, jnp.int32)

    def per_group(g, lv):
        cx = cv[0, pl.ds(g * 16, 16)]
        cy = cv[1, pl.ds(g * 16, 16)]
        cz = cv[2, pl.ds(g * 16, 16)]
        laneoff = iota16 * _LIST_CAP

        def scan_chunk(jc, cnt):
            base = jc * 16
            px16 = xv[0, pl.ds(base, 16)]
            py16 = xv[1, pl.ds(base, 16)]
            pz16 = xv[2, pl.ds(base, 16)]
            for i in range(16):
                dx = px16[i] - cx
                dy = py16[i] - cy
                dz = pz16[i] - cz
                d = dx * dx + dy * dy + dz * dz
                m = d <= r2
                stm = jnp.logical_and(m, cnt < _LIST_CAP)
                plsc.store_scatter(lv, [laneoff + cnt],
                                   jnp.full((16,), base + i, jnp.int32),
                                   mask=stm)
                cnt = cnt + m.astype(jnp.int32)
            return cnt

        cnt = lax.fori_loop(0, N // 16, scan_chunk, zeros16)

        for k in range(NSAMPLE):
            pos = lax.rem(jnp.full((16,), k, jnp.int32), cnt)
            sidx = plsc.load_gather(lv, [laneoff + pos])
            gx = plsc.load_gather(xv, [zeros16, sidx]) - cx
            gy = plsc.load_gather(xv, [ones16, sidx]) - cy
            gz = plsc.load_gather(xv, [twos16, sidx]) - cz
            ov[pl.ds(((k * 3 + 0) * _GROUPS_PER_TILE + g) * 16, 16)] = gx
            ov[pl.ds(((k * 3 + 1) * _GROUPS_PER_TILE + g) * 16, 16)] = gy
            ov[pl.ds(((k * 3 + 2) * _GROUPS_PER_TILE + g) * 16, 16)] = gz

    for g in range(_GROUPS_PER_TILE):
        per_group(g, lv)

    pltpu.sync_copy(ov, out_hbm.at[wid])


def _ball_group(xyz_bcn, cent_bcn):
    mesh = plsc.VectorSubcoreMesh(core_axis_name="c", subcore_axis_name="s",
                                  num_cores=2, num_subcores=16)
    return pl.kernel(
        _ball_group_body,
        out_type=jax.ShapeDtypeStruct(
            (_NTILES, NSAMPLE * 3 * _CENT_PER_TILE), jnp.float32),
        mesh=mesh,
        compiler_params=pltpu.CompilerParams(needs_layout_passes=False),
        scratch_types=[
            pltpu.VMEM((3, N), jnp.float32),
            pltpu.VMEM((3, _CENT_PER_TILE), jnp.float32),
            pltpu.VMEM((NSAMPLE * 3 * _CENT_PER_TILE,), jnp.float32),
            pltpu.VMEM((16 * _LIST_CAP,), jnp.int32),
        ],
    )(xyz_bcn, cent_bcn)


def _mlp_body(g_ref, a0_ref, c0_ref, a1_ref, c1_ref, out_ref):
    a0 = a0_ref[...]
    c0 = c0_ref[...]
    a1 = a1_ref[...]
    c1 = c1_ref[...]

    def step(s, acc):
        x = g_ref[s]
        h1 = (a0[:, 0:1] * x[0:1, :] + a0[:, 1:2] * x[1:2, :]
              + a0[:, 2:3] * x[2:3, :]) + c0
        h1 = jnp.maximum(h1, 0.0)
        h2 = jax.lax.dot_general(a1, h1, (((1,), (0,)), ((), ())),
                                 preferred_element_type=jnp.float32) + c1
        h2 = jnp.maximum(h2, 0.0)
        return jnp.maximum(acc, h2)

    acc0 = jnp.full((a1.shape[0], B * NPOINT), -jnp.inf, dtype=jnp.float32)
    out_ref[...] = lax.fori_loop(0, NSAMPLE, step, acc0)


def _mlp_maxpool(grouped, a0, c0, a1, c1):
    return pl.pallas_call(
        _mlp_body,
        out_shape=jax.ShapeDtypeStruct((a1.shape[0], B * NPOINT), jnp.float32),
    )(grouped, a0, c0, a1, c1)


def kernel(xyz, W0, b0, gamma0, beta0, mean0, var0,
           W1, b1, gamma1, beta1, mean1, var1):
    xyz = xyz.astype(jnp.float32)
    xyz_cbn = jnp.transpose(xyz, (2, 0, 1))
    xyz_bcn = jnp.transpose(xyz, (0, 2, 1))

    cents = _fps(xyz_cbn)
    new_xyz = jnp.transpose(cents, (2, 0, 1))
    cent_bcn = jnp.transpose(cents, (2, 1, 0))

    gout = _ball_group(xyz_bcn, cent_bcn)
    grouped = jnp.transpose(
        gout.reshape(_NTILES, NSAMPLE, 3, _CENT_PER_TILE),
        (1, 2, 0, 3)).reshape(NSAMPLE, 3, B * NPOINT)

    s0 = gamma0 / jnp.sqrt(var0 + EPS)
    a0 = W0 * s0[:, None]
    c0 = ((b0 - mean0) * s0 + beta0)[:, None]
    s1 = gamma1 / jnp.sqrt(var1 + EPS)
    a1 = W1 * s1[:, None]
    c1 = ((b1 - mean1) * s1 + beta1)[:, None]

    pts = _mlp_maxpool(grouped, a0, c0, a1, c1)
    new_points = jnp.transpose(pts.reshape(pts.shape[0], B, NPOINT), (1, 0, 2))
    return new_xyz, new_points
